# Initial kernel scaffold; baseline (speedup 1.0000x reference)
import jax
import jax.numpy as jnp
from jax.experimental import pallas as pl


def kernel(cls_scores_0, cls_scores_1, cls_scores_2, cls_scores_3, cls_scores_4, bbox_preds_0, bbox_preds_1, bbox_preds_2, bbox_preds_3, bbox_preds_4, score_factors_0, score_factors_1, score_factors_2, score_factors_3, score_factors_4):
    raise NotImplementedError("write your pallas kernel here")


# speedup vs baseline: 1.2126x; 1.2126x over previous
import functools

import jax
import jax.numpy as jnp
from jax.experimental import pallas as pl

_STRIDES = [8, 16, 32, 64, 128]
_SIZES = [(40, 40), (20, 20), (10, 10), (5, 5), (3, 3)]
_NC = 80
_NMS_PRE = 1000
_MAXC = 320.0
_KB = 128
_RB = 128


def _decode_body(w, stride, n, idx_ref, val_ref, bpsf_ref, out_ref, lab_ref):
    idx = idx_ref[:, :]
    keep = idx // _NC
    lab = idx % _NC
    col = jax.lax.broadcasted_iota(jnp.int32, (1, n), 1)
    m = (keep == col).astype(jnp.float32)

    def gather_row(r):
        return jnp.sum(m * bpsf_ref[r:r + 1, :], axis=1, keepdims=True)

    d0 = gather_row(0)
    d1 = gather_row(1)
    d2 = gather_row(2)
    d3 = gather_row(3)
    sf = gather_row(4)
    fx = (keep % w).astype(jnp.float32)
    fy = (keep // w).astype(jnp.float32)
    px = (fx + 0.5) * float(stride)
    py = (fy + 0.5) * float(stride)
    x1 = jnp.clip(px - d0, 0.0, _MAXC)
    y1 = jnp.clip(py - d1, 0.0, _MAXC)
    x2 = jnp.clip(px + d2, 0.0, _MAXC)
    y2 = jnp.clip(py + d3, 0.0, _MAXC)
    sc = val_ref[:, :] * sf
    z = jnp.zeros_like(sc)
    out_ref[:, :] = jnp.concatenate([x1, y1, x2, y2, sc, z, z, z], axis=1)
    lab_ref[:, :] = lab


def _decode_level(lvl, idxs, vals, bpsf):
    h, w = _SIZES[lvl]
    n = h * w
    k = idxs.shape[0]
    k_pad = ((k + _KB - 1) // _KB) * _KB
    idxs_p = jnp.zeros((k_pad, 1), jnp.int32).at[:k, 0].set(idxs)
    vals_p = jnp.zeros((k_pad, 1), jnp.float32).at[:k, 0].set(vals)
    body = functools.partial(_decode_body, w, _STRIDES[lvl], n)
    out, lab = pl.pallas_call(
        body,
        grid=(k_pad // _KB,),
        in_specs=[
            pl.BlockSpec((_KB, 1), lambda i: (i, 0)),
            pl.BlockSpec((_KB, 1), lambda i: (i, 0)),
            pl.BlockSpec((5, n), lambda i: (0, 0)),
        ],
        out_specs=[
            pl.BlockSpec((_KB, 8), lambda i: (i, 0)),
            pl.BlockSpec((_KB, 1), lambda i: (i, 0)),
        ],
        out_shape=[
            jax.ShapeDtypeStruct((k_pad, 8), jnp.float32),
            jax.ShapeDtypeStruct((k_pad, 1), jnp.int32),
        ],
    )(idxs_p, vals_p, bpsf)
    return out[:k, :4], out[:k, 4], lab[:k, 0]


def _nms_body(n_pad, rows_ref, cols_ref, out_ref):
    i = pl.program_id(0)
    rx1 = rows_ref[:, 0:1]
    ry1 = rows_ref[:, 1:2]
    rx2 = rows_ref[:, 2:3]
    ry2 = rows_ref[:, 3:4]
    cx1 = cols_ref[0:1, :]
    cy1 = cols_ref[1:2, :]
    cx2 = cols_ref[2:3, :]
    cy2 = cols_ref[3:4, :]
    ra = (rx2 - rx1) * (ry2 - ry1)
    ca = (cx2 - cx1) * (cy2 - cy1)
    ww = jnp.maximum(jnp.minimum(rx2, cx2) - jnp.maximum(rx1, cx1), 0.0)
    hh = jnp.maximum(jnp.minimum(ry2, cy2) - jnp.maximum(ry1, cy1), 0.0)
    inter = ww * hh
    den = ra + ca - inter + 1e-6
    sup = inter > 0.5 * den
    ri = i * _RB + jax.lax.broadcasted_iota(jnp.int32, (_RB, 1), 0)
    cj = jax.lax.broadcasted_iota(jnp.int32, (1, n_pad), 1)
    m = (cj < ri) & sup
    out_ref[:, :] = jnp.sum(m.astype(jnp.float32), axis=1, keepdims=True)


def _nms_counts(b_sorted):
    n = b_sorted.shape[0]
    n_pad = ((n + _RB - 1) // _RB) * _RB
    rows = jnp.zeros((n_pad, 4), jnp.float32).at[:n].set(b_sorted)
    cols = rows.T
    counts = pl.pallas_call(
        functools.partial(_nms_body, n_pad),
        grid=(n_pad // _RB,),
        in_specs=[
            pl.BlockSpec((_RB, 4), lambda i: (i, 0)),
            pl.BlockSpec((4, n_pad), lambda i: (0, 0)),
        ],
        out_specs=pl.BlockSpec((_RB, 1), lambda i: (i, 0)),
        out_shape=jax.ShapeDtypeStruct((n_pad, 1), jnp.float32),
    )(rows, cols)
    return counts[:n, 0]


def kernel(cls_scores_0, cls_scores_1, cls_scores_2, cls_scores_3, cls_scores_4,
           bbox_preds_0, bbox_preds_1, bbox_preds_2, bbox_preds_3, bbox_preds_4,
           score_factors_0, score_factors_1, score_factors_2, score_factors_3,
           score_factors_4):
    cls_scores = [cls_scores_0, cls_scores_1, cls_scores_2, cls_scores_3, cls_scores_4]
    bbox_preds = [bbox_preds_0, bbox_preds_1, bbox_preds_2, bbox_preds_3, bbox_preds_4]
    score_factors = [score_factors_0, score_factors_1, score_factors_2, score_factors_3, score_factors_4]

    mlvl_boxes, mlvl_scores, mlvl_labels = [], [], []
    for lvl in range(5):
        cls = jax.nn.sigmoid(
            jnp.transpose(cls_scores[lvl][0], (1, 2, 0)).reshape(-1, _NC))
        flat = cls.reshape(-1)
        k = min(_NMS_PRE, flat.shape[0])
        vals, idxs = jax.lax.top_k(flat, k)
        bp = jnp.transpose(bbox_preds[lvl][0], (1, 2, 0)).reshape(-1, 4)
        sf = jax.nn.sigmoid(
            jnp.transpose(score_factors[lvl][0], (1, 2, 0)).reshape(-1))
        bpsf = jnp.concatenate([bp.T, sf[None, :]], axis=0)
        b, s, lab = _decode_level(lvl, idxs, vals, bpsf)
        mlvl_boxes.append(b)
        mlvl_scores.append(s)
        mlvl_labels.append(lab)

    boxes = jnp.concatenate(mlvl_boxes, axis=0)
    scores = jnp.concatenate(mlvl_scores, axis=0)
    labels = jnp.concatenate(mlvl_labels, axis=0)

    max_coord = boxes.max()
    offsets = labels.astype(boxes.dtype) * (max_coord + 1.0)
    boxes_nms = boxes + offsets[:, None]

    order = jnp.argsort(-scores)
    b = jnp.take(boxes_nms, order, axis=0)
    s = jnp.take(scores, order)
    lab = jnp.take(labels, order)
    raw = jnp.take(boxes, order, axis=0)

    counts = _nms_counts(b)
    keep_mask = counts == 0.0
    final_scores = jnp.where(keep_mask, s, 0.0)
    dets = jnp.concatenate([raw, final_scores[:, None]], axis=1)
    return dets, lab, keep_mask


# speedup vs baseline: 1.4145x; 1.1665x over previous
import functools

import jax
import jax.numpy as jnp
from jax.experimental import pallas as pl

_STRIDES = [8, 16, 32, 64, 128]
_SIZES = [(40, 40), (20, 20), (10, 10), (5, 5), (3, 3)]
_NC = 80
_NMS_PRE = 1000
_MAXC = 320.0
_KB = 256
_KPAD = 1024
_NMAX = 1600
_RB = 256


def _decode_body(keep_ref, val_ref, bpsf_ref, out_ref):
    keep = keep_ref[:, :]
    tab = bpsf_ref[0]
    col = jax.lax.broadcasted_iota(jnp.int32, (1, _NMAX), 1)
    m = (keep == col).astype(jnp.float32)

    def gather_row(r):
        return jnp.sum(m * tab[r:r + 1, :], axis=1, keepdims=True)

    d0 = gather_row(0)
    d1 = gather_row(1)
    d2 = gather_row(2)
    d3 = gather_row(3)
    sf = gather_row(4)
    px = gather_row(5)
    py = gather_row(6)
    x1 = jnp.clip(px - d0, 0.0, _MAXC)
    y1 = jnp.clip(py - d1, 0.0, _MAXC)
    x2 = jnp.clip(px + d2, 0.0, _MAXC)
    y2 = jnp.clip(py + d3, 0.0, _MAXC)
    sc = val_ref[:, :] * sf
    z = jnp.zeros_like(sc)
    out_ref[:, :] = jnp.concatenate([x1, y1, x2, y2, sc, z, z, z], axis=1)


def _decode_all(keeps, vals, bpsf):
    blocks_per_lvl = _KPAD // _KB
    out = pl.pallas_call(
        _decode_body,
        grid=(5 * blocks_per_lvl,),
        in_specs=[
            pl.BlockSpec((_KB, 1), lambda i: (i, 0)),
            pl.BlockSpec((_KB, 1), lambda i: (i, 0)),
            pl.BlockSpec((1, 7, _NMAX), lambda i: (i // blocks_per_lvl, 0, 0)),
        ],
        out_specs=pl.BlockSpec((_KB, 8), lambda i: (i, 0)),
        out_shape=jax.ShapeDtypeStruct((5 * _KPAD, 8), jnp.float32),
    )(keeps, vals, bpsf)
    return out


def _nms_body(n_pad, rows_ref, cols_ref, out_ref):
    i = pl.program_id(0)
    rx1 = rows_ref[:, 0:1]
    ry1 = rows_ref[:, 1:2]
    rx2 = rows_ref[:, 2:3]
    ry2 = rows_ref[:, 3:4]
    cx1 = cols_ref[0:1, :]
    cy1 = cols_ref[1:2, :]
    cx2 = cols_ref[2:3, :]
    cy2 = cols_ref[3:4, :]
    ra = (rx2 - rx1) * (ry2 - ry1)
    ca = (cx2 - cx1) * (cy2 - cy1)
    ww = jnp.maximum(jnp.minimum(rx2, cx2) - jnp.maximum(rx1, cx1), 0.0)
    hh = jnp.maximum(jnp.minimum(ry2, cy2) - jnp.maximum(ry1, cy1), 0.0)
    inter = ww * hh
    den = ra + ca - inter + 1e-6
    sup = inter > 0.5 * den
    ri = i * _RB + jax.lax.broadcasted_iota(jnp.int32, (_RB, 1), 0)
    cj = jax.lax.broadcasted_iota(jnp.int32, (1, n_pad), 1)
    m = (cj < ri) & sup
    out_ref[:, :] = jnp.sum(m.astype(jnp.float32), axis=1, keepdims=True)


def _nms_counts(b_sorted):
    n = b_sorted.shape[0]
    n_pad = ((n + _RB - 1) // _RB) * _RB
    rows = jnp.zeros((n_pad, 4), jnp.float32).at[:n].set(b_sorted)
    cols = rows.T
    counts = pl.pallas_call(
        functools.partial(_nms_body, n_pad),
        grid=(n_pad // _RB,),
        in_specs=[
            pl.BlockSpec((_RB, 4), lambda i: (i, 0)),
            pl.BlockSpec((4, n_pad), lambda i: (0, 0)),
        ],
        out_specs=pl.BlockSpec((_RB, 1), lambda i: (i, 0)),
        out_shape=jax.ShapeDtypeStruct((n_pad, 1), jnp.float32),
    )(rows, cols)
    return counts[:n, 0]


def kernel(cls_scores_0, cls_scores_1, cls_scores_2, cls_scores_3, cls_scores_4,
           bbox_preds_0, bbox_preds_1, bbox_preds_2, bbox_preds_3, bbox_preds_4,
           score_factors_0, score_factors_1, score_factors_2, score_factors_3,
           score_factors_4):
    cls_scores = [cls_scores_0, cls_scores_1, cls_scores_2, cls_scores_3, cls_scores_4]
    bbox_preds = [bbox_preds_0, bbox_preds_1, bbox_preds_2, bbox_preds_3, bbox_preds_4]
    score_factors = [score_factors_0, score_factors_1, score_factors_2, score_factors_3, score_factors_4]

    keeps, vals_l, labels_l, bpsf_l, ks = [], [], [], [], []
    for lvl in range(5):
        h, w = _SIZES[lvl]
        n = h * w
        stride = float(_STRIDES[lvl])
        flat = jax.nn.sigmoid(
            jnp.transpose(cls_scores[lvl][0], (1, 2, 0)).reshape(-1, _NC)
        ).reshape(-1)
        k = min(_NMS_PRE, flat.shape[0])
        vals, idxs = jax.lax.top_k(flat, k)
        keep = idxs // _NC
        labels_l.append(idxs % _NC)
        ks.append(k)
        keeps.append(jnp.zeros((_KPAD, 1), jnp.int32).at[:k, 0].set(keep))
        vals_l.append(jnp.zeros((_KPAD, 1), jnp.float32).at[:k, 0].set(vals))
        bp4 = bbox_preds[lvl][0].reshape(4, n)
        sf1 = jax.nn.sigmoid(score_factors[lvl][0].reshape(1, n))
        sx = (jnp.arange(w, dtype=jnp.float32) + 0.5) * stride
        sy = (jnp.arange(h, dtype=jnp.float32) + 0.5) * stride
        xx = jnp.tile(sx, h)[None, :]
        yy = jnp.repeat(sy, w)[None, :]
        tab = jnp.concatenate([bp4, sf1, xx, yy], axis=0)
        bpsf_l.append(jnp.pad(tab, ((0, 0), (0, _NMAX - n)))[None])

    out = _decode_all(
        jnp.concatenate(keeps, axis=0),
        jnp.concatenate(vals_l, axis=0),
        jnp.concatenate(bpsf_l, axis=0),
    )

    boxes = jnp.concatenate(
        [out[l * _KPAD:l * _KPAD + ks[l], :4] for l in range(5)], axis=0)
    scores = jnp.concatenate(
        [out[l * _KPAD:l * _KPAD + ks[l], 4] for l in range(5)], axis=0)
    labels = jnp.concatenate(labels_l, axis=0)

    max_coord = boxes.max()
    offsets = labels.astype(boxes.dtype) * (max_coord + 1.0)
    boxes_nms = boxes + offsets[:, None]

    order = jnp.argsort(-scores)
    b = jnp.take(boxes_nms, order, axis=0)
    s = jnp.take(scores, order)
    lab = jnp.take(labels, order)
    raw = jnp.take(boxes, order, axis=0)

    counts = _nms_counts(b)
    keep_mask = counts == 0.0
    final_scores = jnp.where(keep_mask, s, 0.0)
    dets = jnp.concatenate([raw, final_scores[:, None]], axis=1)
    return dets, lab, keep_mask
